# baseline (speedup 1.0000x reference)
import functools

import jax
import jax.numpy as jnp
import numpy as np
from jax import lax
from jax.experimental import pallas as pl
from jax.experimental.pallas import tpu as pltpu
from jax.experimental.pallas import tpu_sc as plsc

_LO_PRESS = [100.0, 150, 200, 250, 300, 350, 400, 450, 500, 550]
_HI_PRESS = [200.0, 400, 600, 800, 1000, 1200, 1400, 1600, 1800, 2000]
_COM_SPEED = np.array([
    [2000.0, 2000, 2000, 2000, 2000, 2000, 2000, 2000, 2000, 2000],
    [1600, 1600, 1600, 1600, 1600, 1700, 1800, 1900, 2000, 2000],
    [1200, 1200, 1200, 1200, 1200, 1200, 1200, 1200, 1600, 2000],
    [900, 900, 950, 1000, 1050, 1100, 1150, 1200, 1600, 2000],
    [800, 800, 800, 800, 900, 1000, 1100, 1200, 1600, 2000],
    [800, 800, 800, 800, 800, 900, 1050, 1200, 1600, 2000],
    [800, 800, 800, 800, 800, 800, 1000, 1200, 1600, 2000],
    [800, 800, 800, 800, 800, 800, 950, 1200, 1600, 2000],
    [800, 800, 800, 800, 800, 800, 900, 1200, 1600, 2000],
    [800, 800, 800, 800, 800, 800, 850, 1200, 1600, 2000]], dtype=np.float32)

_TFLAT = np.zeros((128,), np.float32)
_TFLAT[:100] = _COM_SPEED.reshape(-1)

_TILE = 4096
_NC, _NS = 2, 16
_NW = _NC * _NS


def _sc_interp_body(lo_hbm, hi_hbm, out_hbm, lo_v, hi_v, tab_v, out_v):
    chunk = out_v.shape[0]
    B = chunk * _NW
    wid = lax.axis_index("s") * _NC + lax.axis_index("c")
    base = wid * chunk
    pltpu.sync_copy(lo_hbm.at[pl.ds(base, chunk)], lo_v)
    pltpu.sync_copy(hi_hbm.at[pl.ds(base, chunk)], hi_v)
    pltpu.sync_copy(lo_hbm.at[pl.ds(B, 128)], tab_v)
    for i in range(chunk // 16):
        lo = lo_v[pl.ds(i * 16, 16)]
        hi = hi_v[pl.ds(i * 16, 16)]
        u1 = (lo - 100.0) * (1.0 / 50.0)
        u2 = (hi - 200.0) * (1.0 / 200.0)
        t1 = u1.astype(jnp.int32)
        t2 = u2.astype(jnp.int32)
        c1 = t1 + jnp.where(t1.astype(jnp.float32) < u1, 1, 0)
        c2 = t2 + jnp.where(t2.astype(jnp.float32) < u2, 1, 0)
        i1 = jnp.clip(c1 - 1, 0, 8)
        i2 = jnp.clip(c2 - 1, 0, 8)
        idx = i1 * 10 + i2
        q11 = plsc.load_gather(tab_v, [idx])
        q12 = plsc.load_gather(tab_v, [idx + 1])
        q21 = plsc.load_gather(tab_v, [idx + 10])
        q22 = plsc.load_gather(tab_v, [idx + 11])
        i1f = i1.astype(jnp.float32)
        i2f = i2.astype(jnp.float32)
        xr = (lo - (100.0 + 50.0 * i1f)) / 50.0
        yr = (hi - (200.0 + 200.0 * i2f)) / 200.0
        r1 = xr * (q21 - q11) + q11
        r2 = xr * (q22 - q12) + q12
        out_v[pl.ds(i * 16, 16)] = yr * (r2 - r1) + r1
    pltpu.sync_copy(out_v, out_hbm.at[pl.ds(base, chunk)])


def _sc_interp(lo_ext, hi):
    B = hi.shape[0]
    chunk = B // _NW
    mesh = plsc.VectorSubcoreMesh(core_axis_name="c", subcore_axis_name="s",
                                  num_cores=_NC, num_subcores=_NS)
    return pl.kernel(
        _sc_interp_body,
        out_type=jax.ShapeDtypeStruct((B,), jnp.float32),
        mesh=mesh,
        compiler_params=pltpu.CompilerParams(needs_layout_passes=False),
        scratch_types=[
            pltpu.VMEM((chunk,), jnp.float32),
            pltpu.VMEM((chunk,), jnp.float32),
            pltpu.VMEM((128,), jnp.float32),
            pltpu.VMEM((chunk,), jnp.float32),
        ],
    )(lo_ext, hi)


def _mlp_kernel(xt_ref, v12t_ref, ball_ref, w31_ref, w41_ref,
                wlast_ref, out_ref):
    bf = jnp.bfloat16
    xt = xt_ref[...]
    b12t = ball_ref[0:512, :]
    b31t = ball_ref[512:768, :]
    b41t = ball_ref[768:1024, :]
    blastt = ball_ref[1024:1026, :]
    h = jnp.tanh(jnp.dot(v12t_ref[...], xt,
                         preferred_element_type=jnp.float32) + b12t)
    h1 = jnp.tanh(jnp.dot(w31_ref[...], h[:256, :],
                          preferred_element_type=jnp.float32) + b31t)
    h2 = jnp.tanh(jnp.dot(w41_ref[...], h[256:, :],
                          preferred_element_type=jnp.float32) + b41t)
    hcat = jnp.concatenate([h1, h2], axis=0)
    o12 = jnp.dot(wlast_ref[...], hcat, preferred_element_type=jnp.float32)
    out_ref[...] = o12 + blastt


def kernel(x, W3_0, b3_0, W3_1, b3_1, W3_2, b3_2,
           W4_0, b4_0, W4_1, b4_1, W4_2, b4_2):
    B = x.shape[0]
    f = jnp.float32
    s1t = np.zeros((6, 7), np.float32)
    for j, c in enumerate([4, 6, 2, 5, 1, 3]):
        s1t[j, c] = 1.0
    s2t = np.zeros((2, 7), np.float32)
    s2t[0, 4] = 1.0; s2t[0, 5] = -1.0
    s2t[1, 3] = 1.0; s2t[1, 2] = -1.0
    v12t = jnp.concatenate(
        [jnp.dot(W3_0, jnp.asarray(s1t)), jnp.dot(W4_0, jnp.asarray(s2t))],
        axis=0)
    ball = jnp.concatenate([b3_0, b4_0, b3_1, b4_1, b3_2, b4_2,
                            jnp.zeros((6,), f)])[:, None]

    z256 = jnp.zeros((1, 256), f)
    wlast = jnp.concatenate(
        [jnp.concatenate([W3_2, z256], axis=1),
         jnp.concatenate([z256, W4_2], axis=1)], axis=0)

    col0 = _sc_interp(jnp.concatenate([x[:, 1], jnp.asarray(_TFLAT)]),
                      x[:, 2])

    out2t = pl.pallas_call(
        _mlp_kernel,
        grid=(B // _TILE,),
        in_specs=[
            pl.BlockSpec((7, _TILE), lambda i: (0, i)),
            pl.BlockSpec((512, 7), lambda i: (0, 0)),
            pl.BlockSpec((1032, 1), lambda i: (0, 0)),
            pl.BlockSpec((256, 256), lambda i: (0, 0)),
            pl.BlockSpec((256, 256), lambda i: (0, 0)),
            pl.BlockSpec((2, 512), lambda i: (0, 0)),
        ],
        out_specs=pl.BlockSpec((2, _TILE), lambda i: (0, i)),
        out_shape=jax.ShapeDtypeStruct((2, B), f),
    )(x.T, v12t, ball, W3_1, W4_1, wlast)

    return jnp.concatenate([col0[None, :], out2t], axis=0).T


# speedup vs baseline: 1.0212x; 1.0212x over previous
import functools

import jax
import jax.numpy as jnp
import numpy as np
from jax import lax
from jax.experimental import pallas as pl
from jax.experimental.pallas import tpu as pltpu
from jax.experimental.pallas import tpu_sc as plsc

_LO_PRESS = [100.0, 150, 200, 250, 300, 350, 400, 450, 500, 550]
_HI_PRESS = [200.0, 400, 600, 800, 1000, 1200, 1400, 1600, 1800, 2000]
_COM_SPEED = np.array([
    [2000.0, 2000, 2000, 2000, 2000, 2000, 2000, 2000, 2000, 2000],
    [1600, 1600, 1600, 1600, 1600, 1700, 1800, 1900, 2000, 2000],
    [1200, 1200, 1200, 1200, 1200, 1200, 1200, 1200, 1600, 2000],
    [900, 900, 950, 1000, 1050, 1100, 1150, 1200, 1600, 2000],
    [800, 800, 800, 800, 900, 1000, 1100, 1200, 1600, 2000],
    [800, 800, 800, 800, 800, 900, 1050, 1200, 1600, 2000],
    [800, 800, 800, 800, 800, 800, 1000, 1200, 1600, 2000],
    [800, 800, 800, 800, 800, 800, 950, 1200, 1600, 2000],
    [800, 800, 800, 800, 800, 800, 900, 1200, 1600, 2000],
    [800, 800, 800, 800, 800, 800, 850, 1200, 1600, 2000]], dtype=np.float32)

_TFLAT = np.zeros((128,), np.float32)
_TFLAT[:100] = _COM_SPEED.reshape(-1)

_TILE = 4096
_NC, _NS = 2, 16
_NW = _NC * _NS


def _sc_interp_body(lo_hbm, hi_hbm, out_hbm, lo_v, hi_v, tab_v, out_v):
    chunk = out_v.shape[0]
    B = chunk * _NW
    wid = lax.axis_index("s") * _NC + lax.axis_index("c")
    base = wid * chunk
    pltpu.sync_copy(lo_hbm.at[pl.ds(base, chunk)], lo_v)
    pltpu.sync_copy(hi_hbm.at[pl.ds(base, chunk)], hi_v)
    pltpu.sync_copy(lo_hbm.at[pl.ds(B, 128)], tab_v)
    for i in range(chunk // 16):
        lo = lo_v[pl.ds(i * 16, 16)]
        hi = hi_v[pl.ds(i * 16, 16)]
        u1 = (lo - 100.0) * (1.0 / 50.0)
        u2 = (hi - 200.0) * (1.0 / 200.0)
        t1 = u1.astype(jnp.int32)
        t2 = u2.astype(jnp.int32)
        c1 = t1 + jnp.where(t1.astype(jnp.float32) < u1, 1, 0)
        c2 = t2 + jnp.where(t2.astype(jnp.float32) < u2, 1, 0)
        i1 = jnp.clip(c1 - 1, 0, 8)
        i2 = jnp.clip(c2 - 1, 0, 8)
        idx = i1 * 10 + i2
        q11 = plsc.load_gather(tab_v, [idx])
        q12 = plsc.load_gather(tab_v, [idx + 1])
        q21 = plsc.load_gather(tab_v, [idx + 10])
        q22 = plsc.load_gather(tab_v, [idx + 11])
        i1f = i1.astype(jnp.float32)
        i2f = i2.astype(jnp.float32)
        xr = (lo - (100.0 + 50.0 * i1f)) / 50.0
        yr = (hi - (200.0 + 200.0 * i2f)) / 200.0
        r1 = xr * (q21 - q11) + q11
        r2 = xr * (q22 - q12) + q12
        out_v[pl.ds(i * 16, 16)] = yr * (r2 - r1) + r1
    pltpu.sync_copy(out_v, out_hbm.at[pl.ds(base, chunk)])


def _sc_interp(lo_ext, hi):
    B = hi.shape[0]
    chunk = B // _NW
    mesh = plsc.VectorSubcoreMesh(core_axis_name="c", subcore_axis_name="s",
                                  num_cores=_NC, num_subcores=_NS)
    return pl.kernel(
        _sc_interp_body,
        out_type=jax.ShapeDtypeStruct((B,), jnp.float32),
        mesh=mesh,
        compiler_params=pltpu.CompilerParams(needs_layout_passes=False),
        scratch_types=[
            pltpu.VMEM((chunk,), jnp.float32),
            pltpu.VMEM((chunk,), jnp.float32),
            pltpu.VMEM((128,), jnp.float32),
            pltpu.VMEM((chunk,), jnp.float32),
        ],
    )(lo_ext, hi)


def _mlp_kernel(xt_ref, v12t_ref, ball_ref, w31_ref, w41_ref,
                w32_ref, w42_ref, out_ref):
    bf = jnp.bfloat16
    xt = xt_ref[...]
    b12t = ball_ref[0:512, :]
    b31t = ball_ref[512:768, :]
    b41t = ball_ref[768:1024, :]
    blastt = ball_ref[1024:1026, :]
    h = jnp.tanh(jnp.dot(v12t_ref[...], xt,
                         preferred_element_type=jnp.float32) + b12t)
    h1 = jnp.tanh(jnp.dot(w31_ref[...], h[:256, :],
                          preferred_element_type=jnp.float32) + b31t)
    h2 = jnp.tanh(jnp.dot(w41_ref[...], h[256:, :],
                          preferred_element_type=jnp.float32) + b41t)
    o1 = jnp.dot(w32_ref[...], h1,
                 preferred_element_type=jnp.float32)
    o2 = jnp.dot(w42_ref[...], h2,
                 preferred_element_type=jnp.float32)
    out_ref[...] = jnp.concatenate([o1, o2], axis=0) + blastt


def kernel(x, W3_0, b3_0, W3_1, b3_1, W3_2, b3_2,
           W4_0, b4_0, W4_1, b4_1, W4_2, b4_2):
    B = x.shape[0]
    f = jnp.float32
    s1t = np.zeros((6, 7), np.float32)
    for j, c in enumerate([4, 6, 2, 5, 1, 3]):
        s1t[j, c] = 1.0
    s2t = np.zeros((2, 7), np.float32)
    s2t[0, 4] = 1.0; s2t[0, 5] = -1.0
    s2t[1, 3] = 1.0; s2t[1, 2] = -1.0
    v12t = jnp.concatenate(
        [jnp.dot(W3_0, jnp.asarray(s1t)), jnp.dot(W4_0, jnp.asarray(s2t))],
        axis=0)
    ball = jnp.concatenate([b3_0, b4_0, b3_1, b4_1, b3_2, b4_2,
                            jnp.zeros((6,), f)])[:, None]

    col0 = _sc_interp(jnp.concatenate([x[:, 1], jnp.asarray(_TFLAT)]),
                      x[:, 2])

    out2t = pl.pallas_call(
        _mlp_kernel,
        grid=(B // _TILE,),
        in_specs=[
            pl.BlockSpec((7, _TILE), lambda i: (0, i)),
            pl.BlockSpec((512, 7), lambda i: (0, 0)),
            pl.BlockSpec((1032, 1), lambda i: (0, 0)),
            pl.BlockSpec((256, 256), lambda i: (0, 0)),
            pl.BlockSpec((256, 256), lambda i: (0, 0)),
            pl.BlockSpec((1, 256), lambda i: (0, 0)),
            pl.BlockSpec((1, 256), lambda i: (0, 0)),
        ],
        out_specs=pl.BlockSpec((2, _TILE), lambda i: (0, i)),
        out_shape=jax.ShapeDtypeStruct((2, B), f),
    )(x.T, v12t, ball, W3_1, W4_1, W3_2, W4_2)

    return jnp.concatenate([col0[None, :], out2t], axis=0).T
